# Initial kernel scaffold; baseline (speedup 1.0000x reference)
import jax
import jax.numpy as jnp
from jax.experimental import pallas as pl


def kernel(x, edge_index, edge_type, edge_norm, weight, bias):
    raise NotImplementedError("write your pallas kernel here")


# speedup vs baseline: 7.9103x; 7.9103x over previous
import jax
import jax.numpy as jnp
from jax import lax
from jax.experimental import pallas as pl
from jax.experimental.pallas import tpu as pltpu
from jax.experimental.pallas import tpu_sc as plsc

N_NODES = 10000
N_EDGES = 320000
FEAT = 128
NUM_RELS = 8

NUM_SUBCORES = 16
EPW = N_EDGES // NUM_SUBCORES
CHUNK = 80
NCHUNK = EPW // CHUNK
ROWS_SH = 10240
RPT = ROWS_SH // NUM_SUBCORES
RCH = RPT // CHUNK
CNTW = 16


def _matmul_body(x_ref, w_ref, b_ref, y_ref):
    y_ref[...] = (
        jnp.dot(x_ref[...], w_ref[0], preferred_element_type=jnp.float32)
        + b_ref[0]
    )


def _build_y(x, weight, bias):
    bn = 2000
    nb = N_NODES // bn
    return pl.pallas_call(
        _matmul_body,
        grid=(NUM_RELS, nb),
        in_specs=[
            pl.BlockSpec((bn, FEAT), lambda r, i: (i, 0)),
            pl.BlockSpec((1, FEAT, FEAT), lambda r, i: (r, 0, 0)),
            pl.BlockSpec((1, 1, FEAT), lambda r, i: (r, 0, 0)),
        ],
        out_specs=pl.BlockSpec((bn, FEAT), lambda r, i: (r * nb + i, 0)),
        out_shape=jax.ShapeDtypeStruct((NUM_RELS * N_NODES, FEAT), jnp.float32),
    )(x, weight, bias.reshape(NUM_RELS, 1, FEAT))


def _edge_body(y_hbm, src_hbm, rel_hbm, dst_hbm, norm_hbm,
               agg_out, cnt_out,
               src_c, rel_c, norm_c, gidx_c, dst_c, cidx_c, rows, oneh,
               agg_sh, cnt_sh, sem):
    s = lax.axis_index("s")
    iota16 = lax.iota(jnp.int32, 16)
    zero16 = jnp.zeros((16,), jnp.float32)
    one16 = jnp.ones((16,), jnp.float32)

    def fill_ids(base):
        for m in range(CHUNK // 16):
            gidx_c[pl.ds(m * 16, 16)] = jnp.full((16,), base + m * 16,
                                                 jnp.int32) + iota16

    def zfill(i, _):
        for t in range(FEAT // 16):
            rows[i, pl.ds(t * 16, 16)] = zero16
            oneh[i, pl.ds(t * 16, 16)] = zero16
        return 0
    lax.fori_loop(0, CHUNK, zfill, 0)

    for k in range(RCH):
        fill_ids(s * RPT + k * CHUNK)
        pltpu.sync_copy(rows, agg_sh.at[gidx_c])

    @pl.when(s == 0)
    def _():
        fill_ids(0)
        pltpu.sync_copy(rows, cnt_sh.at[gidx_c])

    plsc.subcore_barrier()

    def chunk_body(j, _):
        eoff = s * EPW + j * CHUNK
        pltpu.sync_copy(src_hbm.at[pl.ds(eoff, CHUNK)], src_c)
        pltpu.sync_copy(rel_hbm.at[pl.ds(eoff, CHUNK)], rel_c)
        pltpu.sync_copy(dst_hbm.at[pl.ds(eoff, CHUNK)], dst_c)
        pltpu.sync_copy(norm_hbm.at[pl.ds(eoff, CHUNK)], norm_c)
        for k in range(CHUNK // 16):
            gidx_c[pl.ds(k * 16, 16)] = (
                rel_c[pl.ds(k * 16, 16)] * N_NODES + src_c[pl.ds(k * 16, 16)]
            )
        pltpu.async_copy(y_hbm.at[gidx_c], rows, sem).wait()

        def scale(g, _):
            nv = norm_c[pl.ds(g * 16, 16)]
            for l in range(16):
                nb = jnp.full((16,), nv[l], jnp.float32)
                e = g * 16 + l
                for t in range(FEAT // 16):
                    rows[e, pl.ds(t * 16, 16)] = rows[e, pl.ds(t * 16, 16)] * nb
            return 0
        lax.fori_loop(0, CHUNK // 16, scale, 0)

        pltpu.sync_copy(rows, agg_sh.at[dst_c], add=True)

        for k in range(CHUNK // 16):
            cidx_c[pl.ds(k * 16, 16)] = lax.shift_right_logical(
                dst_c[pl.ds(k * 16, 16)], 7)

        def hset(g, _):
            dv = dst_c[pl.ds(g * 16, 16)]
            for l in range(16):
                d = dv[l]
                lane = jnp.full((16,), d & 15, jnp.int32)
                word = lax.shift_right_logical(d, 4) & 7
                oneh[g * 16 + l, pl.ds(word * 16, 16)] = jnp.where(
                    iota16 == lane, one16, zero16)
            return 0
        lax.fori_loop(0, CHUNK // 16, hset, 0)

        pltpu.sync_copy(oneh, cnt_sh.at[cidx_c], add=True)

        def hclr(g, _):
            dv = dst_c[pl.ds(g * 16, 16)]
            for l in range(16):
                d = dv[l]
                word = lax.shift_right_logical(d, 4) & 7
                oneh[g * 16 + l, pl.ds(word * 16, 16)] = zero16
            return 0
        lax.fori_loop(0, CHUNK // 16, hclr, 0)
        return 0
    lax.fori_loop(0, NCHUNK, chunk_body, 0)

    plsc.subcore_barrier()

    for k in range(RCH):
        base = s * RPT + k * CHUNK
        fill_ids(base)
        pltpu.async_copy(agg_sh.at[gidx_c], rows, sem).wait()
        pltpu.sync_copy(rows, agg_out.at[pl.ds(base, CHUNK)])

    @pl.when(s == 0)
    def _():
        fill_ids(0)
        pltpu.async_copy(cnt_sh.at[gidx_c], oneh, sem).wait()
        pltpu.sync_copy(oneh, cnt_out)


def _edge_phase(y, src, rel, dst, norm):
    mesh = plsc.VectorSubcoreMesh(
        core_axis_name="c", subcore_axis_name="s", num_cores=1
    )
    return pl.kernel(
        _edge_body,
        out_type=(
            pltpu.HBM((ROWS_SH, FEAT), jnp.float32),
            pltpu.HBM((CHUNK, FEAT), jnp.float32),
        ),
        mesh=mesh,
        scratch_types=[
            pltpu.VMEM((CHUNK,), jnp.int32),
            pltpu.VMEM((CHUNK,), jnp.int32),
            pltpu.VMEM((CHUNK,), jnp.float32),
            pltpu.VMEM((CHUNK,), jnp.int32),
            pltpu.VMEM((CHUNK,), jnp.int32),
            pltpu.VMEM((CHUNK,), jnp.int32),
            pltpu.VMEM((CHUNK, FEAT), jnp.float32),
            pltpu.VMEM((CHUNK, FEAT), jnp.float32),
            pltpu.VMEM_SHARED((ROWS_SH, FEAT), jnp.float32),
            pltpu.VMEM_SHARED((CHUNK, FEAT), jnp.float32),
            pltpu.SemaphoreType.DMA,
        ],
    )(y, src, rel, dst, norm)


def _combine_body(agg_ref, cnt_ref, x_ref, h_ref):
    h_ref[...] = jnp.where(cnt_ref[...] > 0.0, agg_ref[...], x_ref[...])


def _combine(aggs, cnts, x):
    bn = 2048
    nb = ROWS_SH // bn
    xp = jnp.pad(x, ((0, ROWS_SH - N_NODES), (0, 0)))
    cnts_col = cnts.reshape(ROWS_SH, 1)
    h = pl.pallas_call(
        _combine_body,
        grid=(nb,),
        in_specs=[
            pl.BlockSpec((bn, FEAT), lambda i: (i, 0)),
            pl.BlockSpec((bn, 1), lambda i: (i, 0)),
            pl.BlockSpec((bn, FEAT), lambda i: (i, 0)),
        ],
        out_specs=pl.BlockSpec((bn, FEAT), lambda i: (i, 0)),
        out_shape=jax.ShapeDtypeStruct((ROWS_SH, FEAT), jnp.float32),
    )(aggs, cnts_col, xp)
    return h[:N_NODES]


def kernel(x, edge_index, edge_type, edge_norm, weight, bias):
    src = edge_index[0].astype(jnp.int32)
    dst = edge_index[1].astype(jnp.int32)
    rel = edge_type.astype(jnp.int32)
    norm = edge_norm.reshape(-1).astype(jnp.float32)
    y = _build_y(x, weight, bias)
    aggs, cnts = _edge_phase(y, src, rel, dst, norm)
    return _combine(aggs, cnts, x)


# speedup vs baseline: 15.1421x; 1.9142x over previous
import jax
import jax.numpy as jnp
from jax import lax
from jax.experimental import pallas as pl
from jax.experimental.pallas import tpu as pltpu
from jax.experimental.pallas import tpu_sc as plsc

N_NODES = 10000
N_EDGES = 320000
FEAT = 128
NUM_RELS = 8

NUM_SUBCORES = 16
EPW = N_EDGES // NUM_SUBCORES
CHUNK = 80
NCHUNK = EPW // CHUNK
ROWS_SH = 10240
RPT = ROWS_SH // NUM_SUBCORES
RCH = RPT // CHUNK
CNTW = 16


def _matmul_body(x_ref, w_ref, b_ref, y_ref):
    y_ref[...] = (
        jnp.dot(x_ref[...], w_ref[0], preferred_element_type=jnp.float32)
        + b_ref[0]
    )


def _build_y(x, weight, bias):
    bn = 2000
    nb = N_NODES // bn
    return pl.pallas_call(
        _matmul_body,
        grid=(NUM_RELS, nb),
        in_specs=[
            pl.BlockSpec((bn, FEAT), lambda r, i: (i, 0)),
            pl.BlockSpec((1, FEAT, FEAT), lambda r, i: (r, 0, 0)),
            pl.BlockSpec((1, 1, FEAT), lambda r, i: (r, 0, 0)),
        ],
        out_specs=pl.BlockSpec((bn, FEAT), lambda r, i: (r * nb + i, 0)),
        out_shape=jax.ShapeDtypeStruct((NUM_RELS * N_NODES, FEAT), jnp.float32),
    )(x, weight, bias.reshape(NUM_RELS, 1, FEAT))


def _edge_body(y_hbm, src_hbm, rel_hbm, dst_hbm, norm_hbm,
               agg_out, cnt_out,
               src_a, rel_a, norm_a, gidx_a, dst_a,
               src_b, rel_b, norm_b, gidx_b, dst_b,
               rows_a, rows_b, grid,
               agg_sh, cnt_sh, semm, sema, semb, semsa, semsb):
    s = lax.axis_index("s")
    iota16 = lax.iota(jnp.int32, 16)
    zero16 = jnp.zeros((16,), jnp.float32)
    one16 = jnp.ones((16,), jnp.float32)

    def fill_ids(idref, base):
        for m in range(CHUNK // 16):
            idref[pl.ds(m * 16, 16)] = jnp.full((16,), base + m * 16,
                                                jnp.int32) + iota16

    def zfill(i, _):
        for t in range(FEAT // 16):
            rows_a[i, pl.ds(t * 16, 16)] = zero16
            grid[i, pl.ds(t * 16, 16)] = zero16
        return 0
    lax.fori_loop(0, CHUNK, zfill, 0)

    for k in range(RCH):
        fill_ids(gidx_a, s * RPT + k * CHUNK)
        pltpu.sync_copy(rows_a, agg_sh.at[gidx_a])

    @pl.when(s == 0)
    def _():
        fill_ids(gidx_a, 0)
        pltpu.sync_copy(rows_a, cnt_sh.at[gidx_a])

    plsc.subcore_barrier()

    def halfchunk(src_c, rel_c, norm_c, gidx_c, dst_c, rows, semg, sems):
        def scale(g, _):
            nv = norm_c[pl.ds(g * 16, 16)]
            for l in range(16):
                nb = jnp.full((16,), nv[l], jnp.float32)
                e = g * 16 + l
                for t in range(FEAT // 16):
                    rows[e, pl.ds(t * 16, 16)] = rows[e, pl.ds(t * 16, 16)] * nb
            return 0
        lax.fori_loop(0, CHUNK // 16, scale, 0)

        def hacc(g, _):
            dv = dst_c[pl.ds(g * 16, 16)]
            for l in range(16):
                d = dv[l]
                r = lax.shift_right_logical(d, 7)
                lane = jnp.full((16,), d & 15, jnp.int32)
                word = lax.shift_right_logical(d, 4) & 7
                grid[r, pl.ds(word * 16, 16)] = (
                    grid[r, pl.ds(word * 16, 16)]
                    + jnp.where(iota16 == lane, one16, zero16))
            return 0
        lax.fori_loop(0, CHUNK // 16, hacc, 0)

        return pltpu.async_copy(rows, agg_sh.at[dst_c], sems, add=True)

    def pair_body(j, _):
        eoff_a = s * EPW + (2 * j) * CHUNK
        eoff_b = eoff_a + CHUNK
        ma = [pltpu.async_copy(src_hbm.at[pl.ds(eoff_a, CHUNK)], src_a, semm),
              pltpu.async_copy(rel_hbm.at[pl.ds(eoff_a, CHUNK)], rel_a, semm),
              pltpu.async_copy(dst_hbm.at[pl.ds(eoff_a, CHUNK)], dst_a, semm),
              pltpu.async_copy(norm_hbm.at[pl.ds(eoff_a, CHUNK)], norm_a, semm),
              pltpu.async_copy(src_hbm.at[pl.ds(eoff_b, CHUNK)], src_b, semm),
              pltpu.async_copy(rel_hbm.at[pl.ds(eoff_b, CHUNK)], rel_b, semm),
              pltpu.async_copy(dst_hbm.at[pl.ds(eoff_b, CHUNK)], dst_b, semm),
              pltpu.async_copy(norm_hbm.at[pl.ds(eoff_b, CHUNK)], norm_b, semm)]
        for c in ma:
            c.wait()
        for k in range(CHUNK // 16):
            gidx_a[pl.ds(k * 16, 16)] = (
                rel_a[pl.ds(k * 16, 16)] * N_NODES + src_a[pl.ds(k * 16, 16)])
            gidx_b[pl.ds(k * 16, 16)] = (
                rel_b[pl.ds(k * 16, 16)] * N_NODES + src_b[pl.ds(k * 16, 16)])
        ga = pltpu.async_copy(y_hbm.at[gidx_a], rows_a, sema)
        gb = pltpu.async_copy(y_hbm.at[gidx_b], rows_b, semb)
        ga.wait()
        sa = halfchunk(src_a, rel_a, norm_a, gidx_a, dst_a, rows_a, sema, semsa)
        gb.wait()
        sb = halfchunk(src_b, rel_b, norm_b, gidx_b, dst_b, rows_b, semb, semsb)
        sa.wait()
        sb.wait()
        return 0
    lax.fori_loop(0, NCHUNK // 2, pair_body, 0)

    fill_ids(gidx_a, 0)
    pltpu.sync_copy(grid, cnt_sh.at[gidx_a], add=True)

    plsc.subcore_barrier()

    for k in range(RCH):
        base = s * RPT + k * CHUNK
        fill_ids(gidx_a, base)
        pltpu.async_copy(agg_sh.at[gidx_a], rows_a, sema).wait()
        pltpu.sync_copy(rows_a, agg_out.at[pl.ds(base, CHUNK)])

    @pl.when(s == 0)
    def _():
        fill_ids(gidx_a, 0)
        pltpu.async_copy(cnt_sh.at[gidx_a], rows_a, sema).wait()
        pltpu.sync_copy(rows_a, cnt_out)


def _edge_phase(y, src, rel, dst, norm):
    mesh = plsc.VectorSubcoreMesh(
        core_axis_name="c", subcore_axis_name="s", num_cores=1
    )
    return pl.kernel(
        _edge_body,
        out_type=(
            pltpu.HBM((ROWS_SH, FEAT), jnp.float32),
            pltpu.HBM((CHUNK, FEAT), jnp.float32),
        ),
        mesh=mesh,
        scratch_types=[
            pltpu.VMEM((CHUNK,), jnp.int32),
            pltpu.VMEM((CHUNK,), jnp.int32),
            pltpu.VMEM((CHUNK,), jnp.float32),
            pltpu.VMEM((CHUNK,), jnp.int32),
            pltpu.VMEM((CHUNK,), jnp.int32),
            pltpu.VMEM((CHUNK,), jnp.int32),
            pltpu.VMEM((CHUNK,), jnp.int32),
            pltpu.VMEM((CHUNK,), jnp.float32),
            pltpu.VMEM((CHUNK,), jnp.int32),
            pltpu.VMEM((CHUNK,), jnp.int32),
            pltpu.VMEM((CHUNK, FEAT), jnp.float32),
            pltpu.VMEM((CHUNK, FEAT), jnp.float32),
            pltpu.VMEM((CHUNK, FEAT), jnp.float32),
            pltpu.VMEM_SHARED((ROWS_SH, FEAT), jnp.float32),
            pltpu.VMEM_SHARED((CHUNK, FEAT), jnp.float32),
            pltpu.SemaphoreType.DMA,
            pltpu.SemaphoreType.DMA,
            pltpu.SemaphoreType.DMA,
            pltpu.SemaphoreType.DMA,
            pltpu.SemaphoreType.DMA,
        ],
    )(y, src, rel, dst, norm)


def _combine_body(agg_ref, cnt_ref, x_ref, h_ref):
    h_ref[...] = jnp.where(cnt_ref[...] > 0.0, agg_ref[...], x_ref[...])


def _combine(aggs, cnts, x):
    bn = 2048
    nb = ROWS_SH // bn
    xp = jnp.pad(x, ((0, ROWS_SH - N_NODES), (0, 0)))
    cnts_col = cnts.reshape(ROWS_SH, 1)
    h = pl.pallas_call(
        _combine_body,
        grid=(nb,),
        in_specs=[
            pl.BlockSpec((bn, FEAT), lambda i: (i, 0)),
            pl.BlockSpec((bn, 1), lambda i: (i, 0)),
            pl.BlockSpec((bn, FEAT), lambda i: (i, 0)),
        ],
        out_specs=pl.BlockSpec((bn, FEAT), lambda i: (i, 0)),
        out_shape=jax.ShapeDtypeStruct((ROWS_SH, FEAT), jnp.float32),
    )(aggs, cnts_col, xp)
    return h[:N_NODES]


def kernel(x, edge_index, edge_type, edge_norm, weight, bias):
    src = edge_index[0].astype(jnp.int32)
    dst = edge_index[1].astype(jnp.int32)
    rel = edge_type.astype(jnp.int32)
    norm = edge_norm.reshape(-1).astype(jnp.float32)
    y = _build_y(x, weight, bias)
    aggs, cnts = _edge_phase(y, src, rel, dst, norm)
    return _combine(aggs, cnts, x)


# speedup vs baseline: 16.4619x; 1.0872x over previous
import jax
import jax.numpy as jnp
from jax import lax
from jax.experimental import pallas as pl
from jax.experimental.pallas import tpu as pltpu
from jax.experimental.pallas import tpu_sc as plsc

N_NODES = 10000
N_EDGES = 320000
FEAT = 128
NUM_RELS = 8

NUM_SUBCORES = 16
EPW = N_EDGES // NUM_SUBCORES
CHUNK = 80
NCHUNK = EPW // CHUNK
ROWS_SH = 10240
RPT = ROWS_SH // NUM_SUBCORES
RCH = RPT // CHUNK
CNTW = 16


def _matmul_body(x_ref, w_ref, b_ref, y_ref):
    y_ref[...] = (
        jnp.dot(x_ref[...], w_ref[0], preferred_element_type=jnp.float32)
        + b_ref[0]
    )


def _build_y(x, weight, bias):
    bn = 2000
    nb = N_NODES // bn
    return pl.pallas_call(
        _matmul_body,
        grid=(NUM_RELS, nb),
        in_specs=[
            pl.BlockSpec((bn, FEAT), lambda r, i: (i, 0)),
            pl.BlockSpec((1, FEAT, FEAT), lambda r, i: (r, 0, 0)),
            pl.BlockSpec((1, 1, FEAT), lambda r, i: (r, 0, 0)),
        ],
        out_specs=pl.BlockSpec((bn, FEAT), lambda r, i: (r * nb + i, 0)),
        out_shape=jax.ShapeDtypeStruct((NUM_RELS * N_NODES, FEAT), jnp.float32),
    )(x, weight, bias.reshape(NUM_RELS, 1, FEAT))


def _edge_body(y_hbm, src_hbm, rel_hbm, dst_hbm, norm_hbm,
               agg_out, cnt_out,
               src_a, rel_a, norm_a, gidx_a, dst_a,
               src_b, rel_b, norm_b, gidx_b, dst_b,
               dsts_a, dsts_b, rows_a, rows_b, grid,
               agg_sh, cnt_sh, semm, sema, semb, semsa, semsb):
    s = lax.axis_index("s")
    iota16 = lax.iota(jnp.int32, 16)
    zero16 = jnp.zeros((16,), jnp.float32)
    one16 = jnp.ones((16,), jnp.float32)

    def fill_ids(idref, base):
        for m in range(CHUNK // 16):
            idref[pl.ds(m * 16, 16)] = jnp.full((16,), base + m * 16,
                                                jnp.int32) + iota16

    def zfill(i, _):
        for t in range(FEAT // 16):
            rows_a[i, pl.ds(t * 16, 16)] = zero16
            rows_b[i, pl.ds(t * 16, 16)] = zero16
            grid[i, pl.ds(t * 16, 16)] = zero16
        return 0
    lax.fori_loop(0, CHUNK, zfill, 0)

    for k in range(RCH):
        fill_ids(gidx_a, s * RPT + k * CHUNK)
        pltpu.sync_copy(rows_a, agg_sh.at[gidx_a])

    @pl.when(s == 0)
    def _():
        fill_ids(gidx_a, 0)
        pltpu.sync_copy(rows_a, cnt_sh.at[gidx_a])

    plsc.subcore_barrier()

    def halfchunk(src_c, rel_c, norm_c, gidx_c, dst_c, dsts, rows, semg, sems):
        def scale(g, _):
            nv = norm_c[pl.ds(g * 16, 16)]
            for l in range(16):
                nb = jnp.full((16,), nv[l], jnp.float32)
                e = g * 16 + l
                for t in range(FEAT // 16):
                    rows[e, pl.ds(t * 16, 16)] = rows[e, pl.ds(t * 16, 16)] * nb
            return 0
        lax.fori_loop(0, CHUNK // 16, scale, 0)

        def hacc(g, _):
            dv = dst_c[pl.ds(g * 16, 16)]
            for l in range(16):
                d = dv[l]
                r = lax.shift_right_logical(d, 7)
                lane = jnp.full((16,), d & 15, jnp.int32)
                word = lax.shift_right_logical(d, 4) & 7
                grid[r, pl.ds(word * 16, 16)] = (
                    grid[r, pl.ds(word * 16, 16)]
                    + jnp.where(iota16 == lane, one16, zero16))
            return 0
        lax.fori_loop(0, CHUNK // 16, hacc, 0)

        for k in range(CHUNK // 16):
            dsts[pl.ds(k * 16, 16)] = dst_c[pl.ds(k * 16, 16)]
        return pltpu.async_copy(rows, agg_sh.at[dsts], sems, add=True)

    fill_ids(dsts_a, 0)
    fill_ids(dsts_b, 0)
    pltpu.async_copy(rows_a, agg_sh.at[dsts_a], semsa, add=True)
    pltpu.async_copy(rows_b, agg_sh.at[dsts_b], semsb, add=True)

    def pair_body(j, _):
        eoff_a = s * EPW + (2 * j) * CHUNK
        eoff_b = eoff_a + CHUNK
        ma = [pltpu.async_copy(src_hbm.at[pl.ds(eoff_a, CHUNK)], src_a, semm),
              pltpu.async_copy(rel_hbm.at[pl.ds(eoff_a, CHUNK)], rel_a, semm),
              pltpu.async_copy(dst_hbm.at[pl.ds(eoff_a, CHUNK)], dst_a, semm),
              pltpu.async_copy(norm_hbm.at[pl.ds(eoff_a, CHUNK)], norm_a, semm),
              pltpu.async_copy(src_hbm.at[pl.ds(eoff_b, CHUNK)], src_b, semm),
              pltpu.async_copy(rel_hbm.at[pl.ds(eoff_b, CHUNK)], rel_b, semm),
              pltpu.async_copy(dst_hbm.at[pl.ds(eoff_b, CHUNK)], dst_b, semm),
              pltpu.async_copy(norm_hbm.at[pl.ds(eoff_b, CHUNK)], norm_b, semm)]
        pltpu.make_async_copy(rows_a, agg_sh.at[dsts_a], semsa).wait()
        pltpu.make_async_copy(rows_b, agg_sh.at[dsts_b], semsb).wait()
        for c in ma:
            c.wait()
        for k in range(CHUNK // 16):
            gidx_a[pl.ds(k * 16, 16)] = (
                rel_a[pl.ds(k * 16, 16)] * N_NODES + src_a[pl.ds(k * 16, 16)])
            gidx_b[pl.ds(k * 16, 16)] = (
                rel_b[pl.ds(k * 16, 16)] * N_NODES + src_b[pl.ds(k * 16, 16)])
        ga = pltpu.async_copy(y_hbm.at[gidx_a], rows_a, sema)
        gb = pltpu.async_copy(y_hbm.at[gidx_b], rows_b, semb)
        ga.wait()
        halfchunk(src_a, rel_a, norm_a, gidx_a, dst_a, dsts_a, rows_a,
                  sema, semsa)
        gb.wait()
        halfchunk(src_b, rel_b, norm_b, gidx_b, dst_b, dsts_b, rows_b,
                  semb, semsb)
        return 0
    lax.fori_loop(0, NCHUNK // 2, pair_body, 0)

    pltpu.make_async_copy(rows_a, agg_sh.at[dsts_a], semsa).wait()
    pltpu.make_async_copy(rows_b, agg_sh.at[dsts_b], semsb).wait()

    fill_ids(gidx_a, 0)
    pltpu.sync_copy(grid, cnt_sh.at[gidx_a], add=True)

    plsc.subcore_barrier()

    for k in range(RCH):
        base = s * RPT + k * CHUNK
        fill_ids(gidx_a, base)
        pltpu.async_copy(agg_sh.at[gidx_a], rows_a, sema).wait()
        pltpu.sync_copy(rows_a, agg_out.at[pl.ds(base, CHUNK)])

    @pl.when(s == 0)
    def _():
        fill_ids(gidx_a, 0)
        pltpu.async_copy(cnt_sh.at[gidx_a], rows_a, sema).wait()
        pltpu.sync_copy(rows_a, cnt_out)


def _edge_phase(y, src, rel, dst, norm):
    mesh = plsc.VectorSubcoreMesh(
        core_axis_name="c", subcore_axis_name="s", num_cores=1
    )
    return pl.kernel(
        _edge_body,
        out_type=(
            pltpu.HBM((ROWS_SH, FEAT), jnp.float32),
            pltpu.HBM((CHUNK, FEAT), jnp.float32),
        ),
        mesh=mesh,
        scratch_types=[
            pltpu.VMEM((CHUNK,), jnp.int32),
            pltpu.VMEM((CHUNK,), jnp.int32),
            pltpu.VMEM((CHUNK,), jnp.float32),
            pltpu.VMEM((CHUNK,), jnp.int32),
            pltpu.VMEM((CHUNK,), jnp.int32),
            pltpu.VMEM((CHUNK,), jnp.int32),
            pltpu.VMEM((CHUNK,), jnp.int32),
            pltpu.VMEM((CHUNK,), jnp.float32),
            pltpu.VMEM((CHUNK,), jnp.int32),
            pltpu.VMEM((CHUNK,), jnp.int32),
            pltpu.VMEM((CHUNK,), jnp.int32),
            pltpu.VMEM((CHUNK,), jnp.int32),
            pltpu.VMEM((CHUNK, FEAT), jnp.float32),
            pltpu.VMEM((CHUNK, FEAT), jnp.float32),
            pltpu.VMEM((CHUNK, FEAT), jnp.float32),
            pltpu.VMEM_SHARED((ROWS_SH, FEAT), jnp.float32),
            pltpu.VMEM_SHARED((CHUNK, FEAT), jnp.float32),
            pltpu.SemaphoreType.DMA,
            pltpu.SemaphoreType.DMA,
            pltpu.SemaphoreType.DMA,
            pltpu.SemaphoreType.DMA,
            pltpu.SemaphoreType.DMA,
        ],
    )(y, src, rel, dst, norm)


def _combine_body(agg_ref, cnt_ref, x_ref, h_ref):
    h_ref[...] = jnp.where(cnt_ref[...] > 0.0, agg_ref[...], x_ref[...])


def _combine(aggs, cnts, x):
    bn = 2048
    nb = ROWS_SH // bn
    xp = jnp.pad(x, ((0, ROWS_SH - N_NODES), (0, 0)))
    cnts_col = cnts.reshape(ROWS_SH, 1)
    h = pl.pallas_call(
        _combine_body,
        grid=(nb,),
        in_specs=[
            pl.BlockSpec((bn, FEAT), lambda i: (i, 0)),
            pl.BlockSpec((bn, 1), lambda i: (i, 0)),
            pl.BlockSpec((bn, FEAT), lambda i: (i, 0)),
        ],
        out_specs=pl.BlockSpec((bn, FEAT), lambda i: (i, 0)),
        out_shape=jax.ShapeDtypeStruct((ROWS_SH, FEAT), jnp.float32),
    )(aggs, cnts_col, xp)
    return h[:N_NODES]


def kernel(x, edge_index, edge_type, edge_norm, weight, bias):
    src = edge_index[0].astype(jnp.int32)
    dst = edge_index[1].astype(jnp.int32)
    rel = edge_type.astype(jnp.int32)
    norm = edge_norm.reshape(-1).astype(jnp.float32)
    y = _build_y(x, weight, bias)
    aggs, cnts = _edge_phase(y, src, rel, dst, norm)
    return _combine(aggs, cnts, x)
